# Initial kernel scaffold; baseline (speedup 1.0000x reference)
import jax
import jax.numpy as jnp
from jax.experimental import pallas as pl


def kernel(pos, pin_offset_x, pin_offset_y, pin2node_map, flat_node2pin_map, flat_node2pin_start_map):
    raise NotImplementedError("write your pallas kernel here")


# speedup vs baseline: 204.1519x; 204.1519x over previous
import functools

import jax
import jax.numpy as jnp
from jax import lax
from jax.experimental import pallas as pl
from jax.experimental.pallas import tpu as pltpu
from jax.experimental.pallas import tpu_sc as plsc

NUM_NODES = 1000000
NUM_PINS = 4000000

NC = 2
NS = 16
NW = NC * NS
LANES = 16

CHUNK = 8000
NUM_CHUNKS = NUM_PINS // CHUNK


def _pin_pos_body(nx_hbm, ny_hbm, idx_hbm, ox_hbm, oy_hbm, out_hbm,
                  idx_v, gx_v, gy_v, ox_v, oy_v, sem_x, sem_y):
    wid = lax.axis_index("s") * NC + lax.axis_index("c")
    n_mine = (NUM_CHUNKS - wid + NW - 1) // NW

    def chunk_body(t, _):
        chunk_id = wid + t * NW
        base = chunk_id * CHUNK
        pltpu.sync_copy(idx_hbm.at[pl.ds(base, CHUNK)], idx_v)
        cx = pltpu.async_copy(nx_hbm.at[idx_v], gx_v, sem_x)
        cy = pltpu.async_copy(ny_hbm.at[idx_v], gy_v, sem_y)
        pltpu.sync_copy(ox_hbm.at[pl.ds(base, CHUNK)], ox_v)
        pltpu.sync_copy(oy_hbm.at[pl.ds(base, CHUNK)], oy_v)
        cx.wait()
        cy.wait()

        def add_body(i, _):
            s = pl.ds(i * LANES, LANES)
            gx_v[s] = gx_v[s] + ox_v[s]
            gy_v[s] = gy_v[s] + oy_v[s]
            return 0

        lax.fori_loop(0, CHUNK // LANES, add_body, 0, unroll=4)
        pltpu.sync_copy(gx_v, out_hbm.at[pl.ds(base, CHUNK)])
        pltpu.sync_copy(gy_v, out_hbm.at[pl.ds(NUM_PINS + base, CHUNK)])
        return 0

    lax.fori_loop(0, n_mine, chunk_body, 0)


@jax.jit
def _pin_pos(node_x, node_y, idx, pin_offset_x, pin_offset_y):
    mesh = plsc.VectorSubcoreMesh(core_axis_name="c", subcore_axis_name="s",
                                  num_cores=NC, num_subcores=NS)
    return pl.kernel(
        _pin_pos_body,
        out_type=jax.ShapeDtypeStruct((2 * NUM_PINS,), jnp.float32),
        mesh=mesh,
        scratch_types=[
            pltpu.VMEM((CHUNK,), jnp.int32),
            pltpu.VMEM((CHUNK,), jnp.float32),
            pltpu.VMEM((CHUNK,), jnp.float32),
            pltpu.VMEM((CHUNK,), jnp.float32),
            pltpu.VMEM((CHUNK,), jnp.float32),
            pltpu.SemaphoreType.DMA,
            pltpu.SemaphoreType.DMA,
        ],
    )(node_x, node_y, idx, pin_offset_x, pin_offset_y)


def kernel(pos, pin_offset_x, pin_offset_y, pin2node_map,
           flat_node2pin_map, flat_node2pin_start_map):
    node_x = pos[:NUM_NODES]
    node_y = pos[NUM_NODES:]
    idx = pin2node_map.astype(jnp.int32)
    return _pin_pos(node_x, node_y, idx, pin_offset_x, pin_offset_y)
